# initial kernel scaffold (unmeasured)
import jax
import jax.numpy as jnp
from jax import lax
from jax.experimental import pallas as pl
from jax.experimental.pallas import tpu as pltpu

N_DEV = 4


def kernel(x, w_mat):
    m_per, k = x.shape
    _, n = w_mat.shape
    n_per = n // N_DEV

    def body(x_ref, w_hbm, out_ref, w_buf, send_ref, recv_ref, amax_ref,
             w_sems, send_sems, recv_sems, amax_send_sems, amax_recv_sems):
        my = lax.axis_index("i")

        barrier = pltpu.get_barrier_semaphore()
        for p in range(N_DEV):
            @pl.when(p != my)
            def _():
                pl.semaphore_signal(
                    barrier, inc=1, device_id=(p,),
                    device_id_type=pl.DeviceIdType.MESH,
                )
        pl.semaphore_wait(barrier, N_DEV - 1)

        def w_copy(j, slot):
            return pltpu.make_async_copy(
                w_hbm.at[:, pl.ds(j * n_per, n_per)],
                w_buf.at[slot],
                w_sems.at[slot],
            )

        def data_rdma(j):
            return pltpu.make_async_remote_copy(
                src_ref=send_ref.at[j],
                dst_ref=recv_ref.at[my],
                send_sem=send_sems.at[j],
                recv_sem=recv_sems.at[my],
                device_id=(j,),
                device_id_type=pl.DeviceIdType.MESH,
            )

        def recv_desc(s):
            return pltpu.make_async_remote_copy(
                src_ref=send_ref.at[s],
                dst_ref=recv_ref.at[s],
                send_sem=send_sems.at[s],
                recv_sem=recv_sems.at[s],
                device_id=(s,),
                device_id_type=pl.DeviceIdType.MESH,
            )

        w_copy(0, 0).start()
        local_amax = jnp.float32(0.0)
        for j in range(N_DEV):
            slot = j % 2
            if j + 1 < N_DEV:
                w_copy(j + 1, (j + 1) % 2).start()
            w_copy(j, slot).wait()
            yj = jnp.maximum(
                jnp.dot(x_ref[...], w_buf[slot],
                        preferred_element_type=jnp.float32),
                0.0,
            )
            send_ref[j] = yj
            local_amax = jnp.maximum(local_amax, jnp.max(yj))

            @pl.when(j != my)
            def _():
                data_rdma(j).start()

        recv_ref[pl.ds(my, 1)] = send_ref[pl.ds(my, 1)]

        amax_ref[pl.ds(my, 1)] = jnp.full((1, 8, 128), local_amax,
                                          dtype=jnp.float32)

        def amax_rdma(p):
            return pltpu.make_async_remote_copy(
                src_ref=amax_ref.at[my],
                dst_ref=amax_ref.at[my],
                send_sem=amax_send_sems.at[p],
                recv_sem=amax_recv_sems.at[my],
                device_id=(p,),
                device_id_type=pl.DeviceIdType.MESH,
            )

        def amax_recv_desc(s):
            return pltpu.make_async_remote_copy(
                src_ref=amax_ref.at[s],
                dst_ref=amax_ref.at[s],
                send_sem=amax_send_sems.at[s],
                recv_sem=amax_recv_sems.at[s],
                device_id=(s,),
                device_id_type=pl.DeviceIdType.MESH,
            )

        for p in range(N_DEV):
            @pl.when(p != my)
            def _():
                amax_rdma(p).start()

        for j in range(N_DEV):
            @pl.when(j != my)
            def _():
                data_rdma(j).wait_send()
                amax_rdma(j).wait_send()
                recv_desc(j).wait_recv()
                amax_recv_desc(j).wait_recv()

        g_amax = jnp.maximum(jnp.max(amax_ref[...]), jnp.float32(1e-30))
        scale = g_amax / 448.0
        inv_scale = 448.0 / g_amax
        for s in range(N_DEV):
            blk = recv_ref[s]
            t = jnp.clip(blk * inv_scale, 0.0, 448.0)
            q = t.astype(jnp.float8_e4m3fn).astype(jnp.float32) * scale
            out_ref[pl.ds(s * m_per, m_per), :] = q

    return pl.pallas_call(
        body,
        out_shape=jax.ShapeDtypeStruct((N_DEV * m_per, n_per), jnp.float32),
        in_specs=[
            pl.BlockSpec(memory_space=pltpu.VMEM),
            pl.BlockSpec(memory_space=pltpu.ANY),
        ],
        out_specs=pl.BlockSpec(memory_space=pltpu.VMEM),
        scratch_shapes=[
            pltpu.VMEM((2, k, n_per), jnp.float32),
            pltpu.VMEM((N_DEV, m_per, n_per), jnp.float32),
            pltpu.VMEM((N_DEV, m_per, n_per), jnp.float32),
            pltpu.VMEM((N_DEV, 8, 128), jnp.float32),
            pltpu.SemaphoreType.DMA((2,)),
            pltpu.SemaphoreType.DMA((N_DEV,)),
            pltpu.SemaphoreType.DMA((N_DEV,)),
            pltpu.SemaphoreType.DMA((N_DEV,)),
            pltpu.SemaphoreType.DMA((N_DEV,)),
        ],
        compiler_params=pltpu.CompilerParams(collective_id=0),
    )(x, w_mat)


# baseline (device time: 88631 ns/iter reference)
import jax
import jax.numpy as jnp
from jax import lax
from jax.experimental import pallas as pl
from jax.experimental.pallas import tpu as pltpu

N_DEV = 4


def kernel(x, w_mat):
    m_per, k = x.shape
    _, n = w_mat.shape
    n_per = n // N_DEV

    def body(x_ref, w_hbm, out_ref, w_buf, send_ref, recv_ref, amax_ref,
             w_sems, send_sems, recv_sems, amax_send_sems, amax_recv_sems):
        my = lax.axis_index("i")

        barrier = pltpu.get_barrier_semaphore()
        for p in range(N_DEV):
            @pl.when(p != my)
            def _():
                pl.semaphore_signal(
                    barrier, inc=1, device_id=(p,),
                    device_id_type=pl.DeviceIdType.MESH,
                )
        pl.semaphore_wait(barrier, N_DEV - 1)

        def w_copy(j, slot):
            return pltpu.make_async_copy(
                w_hbm.at[:, pl.ds(j * n_per, n_per)],
                w_buf.at[slot],
                w_sems.at[slot],
            )

        def data_rdma(j):
            return pltpu.make_async_remote_copy(
                src_ref=send_ref.at[j],
                dst_ref=recv_ref.at[my],
                send_sem=send_sems.at[j],
                recv_sem=recv_sems.at[my],
                device_id=(j,),
                device_id_type=pl.DeviceIdType.MESH,
            )

        def recv_desc(s):
            return pltpu.make_async_remote_copy(
                src_ref=send_ref.at[s],
                dst_ref=recv_ref.at[s],
                send_sem=send_sems.at[s],
                recv_sem=recv_sems.at[s],
                device_id=(s,),
                device_id_type=pl.DeviceIdType.MESH,
            )

        w_copy(0, 0).start()
        local_amax = jnp.float32(0.0)
        for j in range(N_DEV):
            slot = j % 2
            if j + 1 < N_DEV:
                w_copy(j + 1, (j + 1) % 2).start()
            w_copy(j, slot).wait()
            yj = jnp.maximum(
                jnp.dot(x_ref[...], w_buf[slot],
                        preferred_element_type=jnp.float32),
                0.0,
            )
            send_ref[j] = yj
            local_amax = jnp.maximum(local_amax, jnp.max(yj))

            @pl.when(j != my)
            def _():
                data_rdma(j).start()

        recv_ref[pl.ds(my, 1)] = send_ref[pl.ds(my, 1)]

        amax_ref[pl.ds(my, 1)] = jnp.full((1, 8, 128), local_amax,
                                          dtype=jnp.float32)

        def amax_rdma(p):
            return pltpu.make_async_remote_copy(
                src_ref=amax_ref.at[my],
                dst_ref=amax_ref.at[my],
                send_sem=amax_send_sems.at[p],
                recv_sem=amax_recv_sems.at[my],
                device_id=(p,),
                device_id_type=pl.DeviceIdType.MESH,
            )

        def amax_recv_desc(s):
            return pltpu.make_async_remote_copy(
                src_ref=amax_ref.at[s],
                dst_ref=amax_ref.at[s],
                send_sem=amax_send_sems.at[s],
                recv_sem=amax_recv_sems.at[s],
                device_id=(s,),
                device_id_type=pl.DeviceIdType.MESH,
            )

        for p in range(N_DEV):
            @pl.when(p != my)
            def _():
                amax_rdma(p).start()

        for j in range(N_DEV):
            @pl.when(j != my)
            def _():
                data_rdma(j).wait_send()
                amax_rdma(j).wait_send()
                recv_desc(j).wait_recv()
                amax_recv_desc(j).wait_recv()

        g_amax = jnp.maximum(jnp.max(amax_ref[...]), jnp.float32(1e-30))
        scale = g_amax / 448.0
        inv_scale = 448.0 / g_amax
        for s in range(N_DEV):
            blk = recv_ref[s]
            t = jnp.clip(blk * inv_scale, 0.0, 448.0)
            q = t.astype(jnp.float8_e4m3fn).astype(jnp.float32) * scale
            out_ref[pl.ds(s * m_per, m_per), :] = q

    return pl.pallas_call(
        body,
        out_shape=jax.ShapeDtypeStruct((N_DEV * m_per, n_per), jnp.float32),
        in_specs=[
            pl.BlockSpec(memory_space=pltpu.VMEM),
            pl.BlockSpec(memory_space=pltpu.MemorySpace.HBM),
        ],
        out_specs=pl.BlockSpec(memory_space=pltpu.VMEM),
        scratch_shapes=[
            pltpu.VMEM((2, k, n_per), jnp.float32),
            pltpu.VMEM((N_DEV, m_per, n_per), jnp.float32),
            pltpu.VMEM((N_DEV, m_per, n_per), jnp.float32),
            pltpu.VMEM((N_DEV, 8, 128), jnp.float32),
            pltpu.SemaphoreType.DMA((2,)),
            pltpu.SemaphoreType.DMA((N_DEV,)),
            pltpu.SemaphoreType.DMA((N_DEV,)),
            pltpu.SemaphoreType.DMA((N_DEV,)),
            pltpu.SemaphoreType.DMA((N_DEV,)),
        ],
        compiler_params=pltpu.CompilerParams(
            collective_id=0,
            vmem_limit_bytes=60 * 1024 * 1024,
        ),
    )(x, w_mat)


# device time: 58354 ns/iter; 1.5189x vs baseline; 1.5189x over previous
import jax
import jax.numpy as jnp
from jax import lax
from jax.experimental import pallas as pl
from jax.experimental.pallas import tpu as pltpu

N_DEV = 4


def kernel(x, w_mat):
    m_per, k = x.shape
    _, n = w_mat.shape
    n_per = n // N_DEV

    def body(x_ref, w_hbm, out_ref, w_buf, y_ref, sendq_ref, recvq_ref,
             amax_ref, w_sems, send_sems, recv_sems, amax_send_sems,
             amax_recv_sems):
        my = lax.axis_index("i")

        barrier = pltpu.get_barrier_semaphore()
        for p in range(N_DEV):
            @pl.when(p != my)
            def _():
                pl.semaphore_signal(
                    barrier, inc=1, device_id=(p,),
                    device_id_type=pl.DeviceIdType.MESH,
                )
        pl.semaphore_wait(barrier, N_DEV - 1)

        def w_copy(j, slot):
            return pltpu.make_async_copy(
                w_hbm.at[:, pl.ds(j * n_per, n_per)],
                w_buf.at[slot],
                w_sems.at[slot],
            )

        def data_rdma(j):
            return pltpu.make_async_remote_copy(
                src_ref=sendq_ref.at[j],
                dst_ref=recvq_ref.at[my],
                send_sem=send_sems.at[j],
                recv_sem=recv_sems.at[my],
                device_id=(j,),
                device_id_type=pl.DeviceIdType.MESH,
            )

        def recv_desc(s):
            return pltpu.make_async_remote_copy(
                src_ref=sendq_ref.at[s],
                dst_ref=recvq_ref.at[s],
                send_sem=send_sems.at[s],
                recv_sem=recv_sems.at[s],
                device_id=(s,),
                device_id_type=pl.DeviceIdType.MESH,
            )

        w_copy(0, 0).start()
        local_amax = jnp.float32(0.0)
        for j in range(N_DEV):
            slot = j % 2
            if j + 1 < N_DEV:
                w_copy(j + 1, (j + 1) % 2).start()
            w_copy(j, slot).wait()
            yj = jnp.maximum(
                jnp.dot(x_ref[...], w_buf[slot],
                        preferred_element_type=jnp.float32),
                0.0,
            )
            y_ref[j] = yj
            local_amax = jnp.maximum(local_amax, jnp.max(yj))

        amax_ref[pl.ds(my, 1)] = jnp.full((1, 8, 128), local_amax,
                                          dtype=jnp.float32)

        def amax_rdma(p):
            return pltpu.make_async_remote_copy(
                src_ref=amax_ref.at[my],
                dst_ref=amax_ref.at[my],
                send_sem=amax_send_sems.at[p],
                recv_sem=amax_recv_sems.at[my],
                device_id=(p,),
                device_id_type=pl.DeviceIdType.MESH,
            )

        def amax_recv_desc(s):
            return pltpu.make_async_remote_copy(
                src_ref=amax_ref.at[s],
                dst_ref=amax_ref.at[s],
                send_sem=amax_send_sems.at[s],
                recv_sem=amax_recv_sems.at[s],
                device_id=(s,),
                device_id_type=pl.DeviceIdType.MESH,
            )

        for p in range(N_DEV):
            @pl.when(p != my)
            def _():
                amax_rdma(p).start()
        for s in range(N_DEV):
            @pl.when(s != my)
            def _():
                amax_rdma(s).wait_send()
                amax_recv_desc(s).wait_recv()

        g_amax = jnp.maximum(jnp.max(amax_ref[...]), jnp.float32(1e-30))
        scale = g_amax / 448.0
        inv_scale = 448.0 / g_amax

        for j in range(N_DEV):
            t = jnp.clip(y_ref[j] * inv_scale, 0.0, 448.0)
            sendq_ref[j] = t.astype(jnp.float8_e4m3fn)

            @pl.when(j != my)
            def _():
                data_rdma(j).start()

        out_ref[pl.ds(my * m_per, m_per), :] = (
            sendq_ref[pl.ds(my, 1)][0].astype(jnp.float32) * scale
        )

        for s in range(N_DEV):
            @pl.when(s != my)
            def _():
                recv_desc(s).wait_recv()
                out_ref[pl.ds(s * m_per, m_per), :] = (
                    recvq_ref[s].astype(jnp.float32) * scale
                )
        for j in range(N_DEV):
            @pl.when(j != my)
            def _():
                data_rdma(j).wait_send()

    return pl.pallas_call(
        body,
        out_shape=jax.ShapeDtypeStruct((N_DEV * m_per, n_per), jnp.float32),
        in_specs=[
            pl.BlockSpec(memory_space=pltpu.MemorySpace.VMEM),
            pl.BlockSpec(memory_space=pltpu.MemorySpace.HBM),
        ],
        out_specs=pl.BlockSpec(memory_space=pltpu.MemorySpace.VMEM),
        scratch_shapes=[
            pltpu.VMEM((2, k, n_per), jnp.float32),
            pltpu.VMEM((N_DEV, m_per, n_per), jnp.float32),
            pltpu.VMEM((N_DEV, m_per, n_per), jnp.float8_e4m3fn),
            pltpu.VMEM((N_DEV, m_per, n_per), jnp.float8_e4m3fn),
            pltpu.VMEM((N_DEV, 8, 128), jnp.float32),
            pltpu.SemaphoreType.DMA((2,)),
            pltpu.SemaphoreType.DMA((N_DEV,)),
            pltpu.SemaphoreType.DMA((N_DEV,)),
            pltpu.SemaphoreType.DMA((N_DEV,)),
            pltpu.SemaphoreType.DMA((N_DEV,)),
        ],
        compiler_params=pltpu.CompilerParams(
            collective_id=0,
            vmem_limit_bytes=60 * 1024 * 1024,
        ),
    )(x, w_mat)
